# baseline (speedup 1.0000x reference)
import functools

import jax
import jax.numpy as jnp
from jax.experimental import pallas as pl
from jax.experimental.pallas import tpu as pltpu

_K = 5


def _row_stack(a, w_stride, length):
    rows = [a]
    for kh in range(1, _K):
        rows.append(pltpu.roll(a, length - kh * w_stride, axis=1))
    return jnp.concatenate(rows, axis=0)


_CHAINS = 1


def _fwd_kernel(x_ref, w1_ref, b1_ref, w2_ref, b2_ref, wav_ref,
                out_ref, feat_ref, *, s, n, w_stride, keep, c2, c2p):
    xs = x_ref[...]
    per = s // _CHAINS
    length = per * n
    for g in range(_CHAINS):
        base = g * per
        xc = jnp.concatenate([xs[base + i] for i in range(per)], axis=1)
        xc = xc.astype(jnp.bfloat16)

        st1 = _row_stack(xc, w_stride, length)
        st1 = jnp.concatenate(
            [st1, jnp.zeros((1, length), jnp.bfloat16)], axis=0)
        pieces = [st1]
        for kw in range(1, _K):
            pieces.append(pltpu.roll(st1, length - kw, axis=1))
        t1 = jnp.concatenate(pieces, axis=0)
        h1 = jnp.dot(w1_ref[...], t1, preferred_element_type=jnp.float32)
        h1 = jnp.maximum(h1 + b1_ref[...], 0.0).astype(jnp.bfloat16)

        st2 = _row_stack(h1, w_stride, length)
        z = jnp.dot(w2_ref[...], st2, preferred_element_type=jnp.float32)
        acc = z[0:c2]
        for kw in range(1, _K):
            acc = acc + pltpu.roll(z[c2p * kw:c2p * kw + c2],
                                   length - kw, axis=1)
        h2 = jnp.maximum(acc + b2_ref[...], 0.0)

        for i in range(per):
            v = h2[:, i * n:i * n + keep]
            v = v.reshape(c2, keep // w_stride, w_stride)
            feat_ref[base + i, :c2, :, :w_stride] = v

    out_ref[...] = jnp.log(wav_ref[...]) * (1.0 / 20.0)


def kernel(w1, b1, w2, b2, x, wav):
    B, Cin, H, W = x.shape
    C1, C2 = w1.shape[0], w2.shape[0]
    H2, W2 = H - 2 * (_K - 1), W - 2 * (_K - 1)
    n = H * W
    keep = H2 * W
    Nw = wav.shape[-1]
    S = 8
    assert B % S == 0

    K1 = _K * (_K * Cin + 1)
    C2p = -(-C2 // 8) * 8

    x_flat = x.reshape(B, Cin, n)
    wav3 = wav.reshape(B // S, S, Nw)
    w1s = jnp.transpose(w1, (3, 0, 2, 1)).reshape(_K, C1, _K * Cin)
    w1s = jnp.pad(w1s, ((0, 0), (0, 0), (0, 1)))
    w1cat = jnp.transpose(w1s, (1, 0, 2)).reshape(C1, K1).astype(jnp.bfloat16)
    w2s = jnp.transpose(w2, (3, 0, 2, 1)).reshape(_K, C2, _K * C1)
    w2s = jnp.pad(w2s, ((0, 0), (0, C2p - C2), (0, 0)))
    w2cat = w2s.reshape(_K * C2p, _K * C1).astype(jnp.bfloat16)
    b1c = b1.reshape(C1, 1)
    b2c = b2.reshape(C2, 1)

    kern = functools.partial(_fwd_kernel, s=S, n=n, w_stride=W, keep=keep,
                             c2=C2, c2p=C2p)

    out, feat_rows = pl.pallas_call(
        kern,
        grid=(B // S,),
        in_specs=[
            pl.BlockSpec((S, Cin, n), lambda b: (b, 0, 0)),
            pl.BlockSpec((C1, K1), lambda b: (0, 0)),
            pl.BlockSpec((C1, 1), lambda b: (0, 0)),
            pl.BlockSpec((_K * C2p, _K * C1), lambda b: (0, 0)),
            pl.BlockSpec((C2, 1), lambda b: (0, 0)),
            pl.BlockSpec((1, S, Nw), lambda b: (b, 0, 0)),
        ],
        out_specs=[
            pl.BlockSpec((1, S, Nw), lambda b: (b, 0, 0)),
            pl.BlockSpec((S, C2p, H2, 128), lambda b: (b, 0, 0, 0)),
        ],
        out_shape=[
            jax.ShapeDtypeStruct((B // S, S, Nw), wav.dtype),
            jax.ShapeDtypeStruct((B, C2p, H2, 128), jnp.float32),
        ],
        compiler_params=pltpu.CompilerParams(
            dimension_semantics=("parallel",)),
    )(x_flat, w1cat, b1c, w2cat, b2c, wav3)

    return out.reshape(B, Nw), feat_rows[:, :C2, :, :W2]


# speedup vs baseline: 1.1833x; 1.1833x over previous
import functools

import jax
import jax.numpy as jnp
from jax.experimental import pallas as pl
from jax.experimental.pallas import tpu as pltpu

_K = 5


def _row_stack(a, w_stride, length):
    rows = [a]
    for kh in range(1, _K):
        rows.append(pltpu.roll(a, length - kh * w_stride, axis=1))
    return jnp.concatenate(rows, axis=0)


_CHAINS = 1


def _fwd_kernel(x_ref, w1_ref, b1_ref, w2_ref, b2_ref, wav_ref,
                out_ref, feat_ref, *, s, n, w_stride, keep, c2, c2p):
    xs = x_ref[...]
    per = s // _CHAINS
    length = per * n
    for g in range(_CHAINS):
        base = g * per
        xc = jnp.concatenate([xs[base + i] for i in range(per)], axis=1)
        xc = xc.astype(jnp.bfloat16)

        st1 = _row_stack(xc, w_stride, length)
        st1 = jnp.concatenate(
            [st1, jnp.zeros((1, length), jnp.bfloat16)], axis=0)
        pieces = [st1]
        for kw in range(1, _K):
            pieces.append(pltpu.roll(st1, length - kw, axis=1))
        t1 = jnp.concatenate(pieces, axis=0)
        h1 = jnp.dot(w1_ref[...], t1, preferred_element_type=jnp.float32)
        h1 = jnp.maximum(h1 + b1_ref[...], 0.0).astype(jnp.bfloat16)

        st2 = _row_stack(h1, w_stride, length)
        z = jnp.dot(w2_ref[...], st2, preferred_element_type=jnp.float32)
        acc = z[0:c2]
        for kw in range(1, _K):
            acc = acc + pltpu.roll(z[c2p * kw:c2p * kw + c2],
                                   length - kw, axis=1)
        h2 = jnp.maximum(acc + b2_ref[...], 0.0)

        for i in range(per):
            feat_ref[base + i, :c2, :] = h2[:, i * n:i * n + keep]

    out_ref[...] = jnp.log(wav_ref[...]) * (1.0 / 20.0)


def kernel(w1, b1, w2, b2, x, wav):
    B, Cin, H, W = x.shape
    C1, C2 = w1.shape[0], w2.shape[0]
    H2, W2 = H - 2 * (_K - 1), W - 2 * (_K - 1)
    n = H * W
    keep = H2 * W
    Nw = wav.shape[-1]
    S = 8
    assert B % S == 0

    K1 = _K * (_K * Cin + 1)
    C2p = -(-C2 // 8) * 8

    x_flat = x.reshape(B, Cin, n)
    wav3 = wav.reshape(B // S, S, Nw)
    w1s = jnp.transpose(w1, (3, 0, 2, 1)).reshape(_K, C1, _K * Cin)
    w1s = jnp.pad(w1s, ((0, 0), (0, 0), (0, 1)))
    w1cat = jnp.transpose(w1s, (1, 0, 2)).reshape(C1, K1).astype(jnp.bfloat16)
    w2s = jnp.transpose(w2, (3, 0, 2, 1)).reshape(_K, C2, _K * C1)
    w2s = jnp.pad(w2s, ((0, 0), (0, C2p - C2), (0, 0)))
    w2cat = w2s.reshape(_K * C2p, _K * C1).astype(jnp.bfloat16)
    b1c = b1.reshape(C1, 1)
    b2c = b2.reshape(C2, 1)

    kern = functools.partial(_fwd_kernel, s=S, n=n, w_stride=W, keep=keep,
                             c2=C2, c2p=C2p)

    out, feat_rows = pl.pallas_call(
        kern,
        grid=(B // S,),
        in_specs=[
            pl.BlockSpec((S, Cin, n), lambda b: (b, 0, 0)),
            pl.BlockSpec((C1, K1), lambda b: (0, 0)),
            pl.BlockSpec((C1, 1), lambda b: (0, 0)),
            pl.BlockSpec((_K * C2p, _K * C1), lambda b: (0, 0)),
            pl.BlockSpec((C2, 1), lambda b: (0, 0)),
            pl.BlockSpec((1, S, Nw), lambda b: (b, 0, 0)),
        ],
        out_specs=[
            pl.BlockSpec((1, S, Nw), lambda b: (b, 0, 0)),
            pl.BlockSpec((S, C2p, keep), lambda b: (b, 0, 0)),
        ],
        out_shape=[
            jax.ShapeDtypeStruct((B // S, S, Nw), wav.dtype),
            jax.ShapeDtypeStruct((B, C2p, keep), jnp.float32),
        ],
        compiler_params=pltpu.CompilerParams(
            dimension_semantics=("parallel",)),
    )(x_flat, w1cat, b1c, w2cat, b2c, wav3)

    feat = feat_rows.reshape(B, C2p, H2, W)[:, :C2, :, :W2]
    return out.reshape(B, Nw), feat


# speedup vs baseline: 1.2812x; 1.0827x over previous
import functools

import jax
import jax.numpy as jnp
from jax.experimental import pallas as pl
from jax.experimental.pallas import tpu as pltpu

_K = 5


def _row_stack(a, w_stride, length):
    rows = [a]
    for kh in range(1, _K):
        rows.append(pltpu.roll(a, length - kh * w_stride, axis=1))
    return jnp.concatenate(rows, axis=0)


_CHAINS = 1


def _fwd_kernel(x_ref, w1_ref, b1_ref, w2_ref, b2_ref, wav_ref,
                out_ref, feat_ref, *, s, n, w_stride, keep, c2, c2p):
    xs = x_ref[...]
    per = s // _CHAINS
    length = per * n
    for g in range(_CHAINS):
        base = g * per
        xc = jnp.concatenate([xs[base + i] for i in range(per)], axis=1)
        xc = xc.astype(jnp.bfloat16)

        st1 = _row_stack(xc, w_stride, length)
        st1 = jnp.concatenate(
            [st1, jnp.zeros((1, length), jnp.bfloat16)], axis=0)
        pieces = [st1]
        for kw in range(1, _K):
            pieces.append(pltpu.roll(st1, length - kw, axis=1))
        t1 = jnp.concatenate(pieces, axis=0)
        h1 = jnp.dot(w1_ref[...], t1, preferred_element_type=jnp.float32)
        h1 = jnp.maximum(h1 + b1_ref[...], 0.0).astype(jnp.bfloat16)

        st2 = _row_stack(h1, w_stride, length)
        z = jnp.dot(w2_ref[...], st2, preferred_element_type=jnp.float32)
        acc = z[0:c2]
        for kw in range(1, _K):
            acc = acc + pltpu.roll(z[c2p * kw:c2p * kw + c2],
                                   length - kw, axis=1)
        h2 = jnp.maximum(acc + b2_ref[...], 0.0)

        for i in range(per):
            v = h2[:, i * n:i * n + keep]
            v = v.reshape(c2, keep // w_stride, w_stride)
            feat_ref[base + i, :, :, :w_stride] = v

    out_ref[...] = jnp.log(wav_ref[...]) * (1.0 / 20.0)


def kernel(w1, b1, w2, b2, x, wav):
    B, Cin, H, W = x.shape
    C1, C2 = w1.shape[0], w2.shape[0]
    H2, W2 = H - 2 * (_K - 1), W - 2 * (_K - 1)
    n = H * W
    keep = H2 * W
    Nw = wav.shape[-1]
    S = 8
    assert B % S == 0

    K1 = _K * (_K * Cin + 1)
    C2p = -(-C2 // 8) * 8

    x_flat = x.reshape(B, Cin, n)
    wav3 = wav.reshape(B // S, S, Nw)
    w1s = jnp.transpose(w1, (3, 0, 2, 1)).reshape(_K, C1, _K * Cin)
    w1s = jnp.pad(w1s, ((0, 0), (0, 0), (0, 1)))
    w1cat = jnp.transpose(w1s, (1, 0, 2)).reshape(C1, K1).astype(jnp.bfloat16)
    w2s = jnp.transpose(w2, (3, 0, 2, 1)).reshape(_K, C2, _K * C1)
    w2s = jnp.pad(w2s, ((0, 0), (0, C2p - C2), (0, 0)))
    w2cat = w2s.reshape(_K * C2p, _K * C1).astype(jnp.bfloat16)
    b1c = b1.reshape(C1, 1)
    b2c = b2.reshape(C2, 1)

    kern = functools.partial(_fwd_kernel, s=S, n=n, w_stride=W, keep=keep,
                             c2=C2, c2p=C2p)

    out, feat_rows = pl.pallas_call(
        kern,
        grid=(B // S,),
        in_specs=[
            pl.BlockSpec((S, Cin, n), lambda b: (b, 0, 0)),
            pl.BlockSpec((C1, K1), lambda b: (0, 0)),
            pl.BlockSpec((C1, 1), lambda b: (0, 0)),
            pl.BlockSpec((_K * C2p, _K * C1), lambda b: (0, 0)),
            pl.BlockSpec((C2, 1), lambda b: (0, 0)),
            pl.BlockSpec((1, S, Nw), lambda b: (b, 0, 0)),
        ],
        out_specs=[
            pl.BlockSpec((1, S, Nw), lambda b: (b, 0, 0)),
            pl.BlockSpec((S, C2, H2, 128), lambda b: (b, 0, 0, 0)),
        ],
        out_shape=[
            jax.ShapeDtypeStruct((B // S, S, Nw), wav.dtype),
            jax.ShapeDtypeStruct((B, C2, H2, 128), jnp.float32),
        ],
        compiler_params=pltpu.CompilerParams(
            dimension_semantics=("parallel",)),
    )(x_flat, w1cat, b1c, w2cat, b2c, wav3)

    return out.reshape(B, Nw), feat_rows[:, :, :, :W2]


# speedup vs baseline: 1.3264x; 1.0353x over previous
import functools

import jax
import jax.numpy as jnp
from jax.experimental import pallas as pl
from jax.experimental.pallas import tpu as pltpu

_K = 5


def _row_stack(a, w_stride, length):
    rows = [a]
    for kh in range(1, _K):
        rows.append(pltpu.roll(a, length - kh * w_stride, axis=1))
    return jnp.concatenate(rows, axis=0)


_CHAINS = 1


def _fwd_kernel(x_ref, w1_ref, b1_ref, w2_ref, b2_ref, wav_ref,
                out_ref, feat_ref, *, s, n, w_stride, keep, c2, c2p):
    xs = x_ref[...]
    per = s // _CHAINS
    length = per * n
    for g in range(_CHAINS):
        base = g * per
        xc = jnp.concatenate([xs[base + i] for i in range(per)], axis=1)
        xc = xc.astype(jnp.bfloat16)

        st1 = _row_stack(xc, w_stride, length)
        st1 = jnp.concatenate(
            [st1, jnp.zeros((1, length), jnp.bfloat16)], axis=0)
        pieces = [st1]
        for kw in range(1, _K):
            pieces.append(pltpu.roll(st1, length - kw, axis=1))
        t1 = jnp.concatenate(pieces, axis=0)
        h1 = jnp.dot(w1_ref[...], t1, preferred_element_type=jnp.float32)
        h1 = jnp.maximum(h1 + b1_ref[...], 0.0).astype(jnp.bfloat16)

        st2 = _row_stack(h1, w_stride, length)
        for i in range(per):
            c0 = i * n
            zi = jnp.dot(w2_ref[...], st2[:, c0:c0 + n],
                         preferred_element_type=jnp.float32)
            acc = zi[0:c2, 0:keep]
            for kw in range(1, _K):
                acc = acc + zi[c2p * kw:c2p * kw + c2, kw:kw + keep]
            h2i = jnp.maximum(acc + b2_ref[...], 0.0)
            v = h2i.reshape(c2, keep // w_stride, w_stride)
            feat_ref[base + i, :, :, :w_stride] = v

    out_ref[...] = jnp.log(wav_ref[...]) * (1.0 / 20.0)


def kernel(w1, b1, w2, b2, x, wav):
    B, Cin, H, W = x.shape
    C1, C2 = w1.shape[0], w2.shape[0]
    H2, W2 = H - 2 * (_K - 1), W - 2 * (_K - 1)
    n = H * W
    keep = H2 * W
    Nw = wav.shape[-1]
    S = 8
    assert B % S == 0

    K1 = _K * (_K * Cin + 1)
    C2p = -(-C2 // 8) * 8

    x_flat = x.reshape(B, Cin, n)
    wav3 = wav.reshape(B // S, S, Nw)
    w1s = jnp.transpose(w1, (3, 0, 2, 1)).reshape(_K, C1, _K * Cin)
    w1s = jnp.pad(w1s, ((0, 0), (0, 0), (0, 1)))
    w1cat = jnp.transpose(w1s, (1, 0, 2)).reshape(C1, K1).astype(jnp.bfloat16)
    w2s = jnp.transpose(w2, (3, 0, 2, 1)).reshape(_K, C2, _K * C1)
    w2s = jnp.pad(w2s, ((0, 0), (0, C2p - C2), (0, 0)))
    w2cat = w2s.reshape(_K * C2p, _K * C1).astype(jnp.bfloat16)
    b1c = b1.reshape(C1, 1)
    b2c = b2.reshape(C2, 1)

    kern = functools.partial(_fwd_kernel, s=S, n=n, w_stride=W, keep=keep,
                             c2=C2, c2p=C2p)

    out, feat_rows = pl.pallas_call(
        kern,
        grid=(B // S,),
        in_specs=[
            pl.BlockSpec((S, Cin, n), lambda b: (b, 0, 0)),
            pl.BlockSpec((C1, K1), lambda b: (0, 0)),
            pl.BlockSpec((C1, 1), lambda b: (0, 0)),
            pl.BlockSpec((_K * C2p, _K * C1), lambda b: (0, 0)),
            pl.BlockSpec((C2, 1), lambda b: (0, 0)),
            pl.BlockSpec((1, S, Nw), lambda b: (b, 0, 0)),
        ],
        out_specs=[
            pl.BlockSpec((1, S, Nw), lambda b: (b, 0, 0)),
            pl.BlockSpec((S, C2, H2, 128), lambda b: (b, 0, 0, 0)),
        ],
        out_shape=[
            jax.ShapeDtypeStruct((B // S, S, Nw), wav.dtype),
            jax.ShapeDtypeStruct((B, C2, H2, 128), jnp.float32),
        ],
        compiler_params=pltpu.CompilerParams(
            dimension_semantics=("parallel",)),
    )(x_flat, w1cat, b1c, w2cat, b2c, wav3)

    return out.reshape(B, Nw), feat_rows[:, :, :, :W2]


# speedup vs baseline: 1.3577x; 1.0236x over previous
import functools

import jax
import jax.numpy as jnp
from jax.experimental import pallas as pl
from jax.experimental.pallas import tpu as pltpu

_K = 5


def _row_stack(a, w_stride, length):
    rows = [a]
    for kh in range(1, _K):
        rows.append(pltpu.roll(a, length - kh * w_stride, axis=1))
    return jnp.concatenate(rows, axis=0)


_CHAINS = 1


def _fwd_kernel(x_ref, w1_ref, b1_ref, w2_ref, b2_ref, wav_ref,
                out_ref, feat_ref, *, s, n, w_stride, keep, c2, c2p):
    xs = x_ref[...]
    per = s // _CHAINS
    length = per * n
    for g in range(_CHAINS):
        base = g * per
        xc = jnp.concatenate([xs[base + i] for i in range(per)], axis=1)
        xc = xc.astype(jnp.bfloat16)

        st1 = _row_stack(xc, w_stride, length)
        st1 = jnp.concatenate(
            [st1, jnp.zeros((1, length), jnp.bfloat16)], axis=0)
        pieces = [st1]
        for kw in range(1, _K):
            pieces.append(pltpu.roll(st1, length - kw, axis=1))
        t1 = jnp.concatenate(pieces, axis=0)
        h1 = jnp.dot(w1_ref[...], t1, preferred_element_type=jnp.float32)
        h1 = jnp.maximum(h1 + b1_ref[...], 0.0).astype(jnp.bfloat16)

        st2 = _row_stack(h1, w_stride, length)
        for i in range(per):
            c0 = i * n
            zi = jnp.dot(w2_ref[...], st2[:, c0:c0 + n],
                         preferred_element_type=jnp.float32)
            acc = zi[0:c2, 0:keep]
            for kw in range(1, _K):
                acc = acc + zi[c2p * kw:c2p * kw + c2, kw:kw + keep]
            h2i = jnp.maximum(acc + b2_ref[...], 0.0)
            v = h2i.reshape(c2, keep // w_stride, w_stride)
            feat_ref[base + i, :, :, :w_stride] = v

    out_ref[...] = jnp.log(wav_ref[...]) * (1.0 / 20.0)


def kernel(w1, b1, w2, b2, x, wav):
    B, Cin, H, W = x.shape
    C1, C2 = w1.shape[0], w2.shape[0]
    H2, W2 = H - 2 * (_K - 1), W - 2 * (_K - 1)
    n = H * W
    keep = H2 * W
    Nw = wav.shape[-1]
    S = 16
    assert B % S == 0

    K1 = _K * (_K * Cin + 1)
    C2p = -(-C2 // 8) * 8

    x_flat = x.reshape(B, Cin, n)
    wav3 = wav.reshape(B // S, S, Nw)
    w1s = jnp.transpose(w1, (3, 0, 2, 1)).reshape(_K, C1, _K * Cin)
    w1s = jnp.pad(w1s, ((0, 0), (0, 0), (0, 1)))
    w1cat = jnp.transpose(w1s, (1, 0, 2)).reshape(C1, K1).astype(jnp.bfloat16)
    w2s = jnp.transpose(w2, (3, 0, 2, 1)).reshape(_K, C2, _K * C1)
    w2s = jnp.pad(w2s, ((0, 0), (0, C2p - C2), (0, 0)))
    w2cat = w2s.reshape(_K * C2p, _K * C1).astype(jnp.bfloat16)
    b1c = b1.reshape(C1, 1)
    b2c = b2.reshape(C2, 1)

    kern = functools.partial(_fwd_kernel, s=S, n=n, w_stride=W, keep=keep,
                             c2=C2, c2p=C2p)

    out, feat_rows = pl.pallas_call(
        kern,
        grid=(B // S,),
        in_specs=[
            pl.BlockSpec((S, Cin, n), lambda b: (b, 0, 0)),
            pl.BlockSpec((C1, K1), lambda b: (0, 0)),
            pl.BlockSpec((C1, 1), lambda b: (0, 0)),
            pl.BlockSpec((_K * C2p, _K * C1), lambda b: (0, 0)),
            pl.BlockSpec((C2, 1), lambda b: (0, 0)),
            pl.BlockSpec((1, S, Nw), lambda b: (b, 0, 0)),
        ],
        out_specs=[
            pl.BlockSpec((1, S, Nw), lambda b: (b, 0, 0)),
            pl.BlockSpec((S, C2, H2, 128), lambda b: (b, 0, 0, 0)),
        ],
        out_shape=[
            jax.ShapeDtypeStruct((B // S, S, Nw), wav.dtype),
            jax.ShapeDtypeStruct((B, C2, H2, 128), jnp.float32),
        ],
        compiler_params=pltpu.CompilerParams(
            dimension_semantics=("parallel",)),
    )(x_flat, w1cat, b1c, w2cat, b2c, wav3)

    return out.reshape(B, Nw), feat_rows[:, :, :, :W2]


# speedup vs baseline: 1.3590x; 1.0009x over previous
import functools

import jax
import jax.numpy as jnp
from jax.experimental import pallas as pl
from jax.experimental.pallas import tpu as pltpu

_K = 5


def _row_stack(a, w_stride, length):
    rows = [a]
    for kh in range(1, _K):
        rows.append(pltpu.roll(a, length - kh * w_stride, axis=1))
    return jnp.concatenate(rows, axis=0)


_CHAINS = 1


def _fwd_kernel(x_ref, w1_ref, b1_ref, w2_ref, b2_ref, wav_ref,
                out_ref, feat_ref, *, s, n, w_stride, keep, c2, c2p):
    xs = x_ref[...]
    per = s // _CHAINS
    length = per * n
    for g in range(_CHAINS):
        base = g * per
        xc = jnp.concatenate([xs[base + i] for i in range(per)], axis=1)
        xc = xc.astype(jnp.bfloat16)

        st1 = _row_stack(xc, w_stride, length)
        st1 = jnp.concatenate(
            [st1, jnp.zeros((1, length), jnp.bfloat16)], axis=0)
        pieces = [st1]
        for kw in range(1, _K):
            pieces.append(pltpu.roll(st1, length - kw, axis=1))
        t1 = jnp.concatenate(pieces, axis=0)
        h1 = jnp.dot(w1_ref[...], t1, preferred_element_type=jnp.float32)
        h1 = jnp.maximum(h1 + b1_ref[...], 0.0).astype(jnp.bfloat16)

        st2 = _row_stack(h1, w_stride, length)
        for i in range(per):
            c0 = i * n
            zi = jnp.dot(w2_ref[...], st2[:, c0:c0 + n],
                         preferred_element_type=jnp.float32)
            acc = zi[0:c2, 0:keep]
            for kw in range(1, _K):
                acc = acc + zi[c2p * kw:c2p * kw + c2, kw:kw + keep]
            h2i = jnp.maximum(acc + b2_ref[...], 0.0)
            v = h2i.reshape(c2, keep // w_stride, w_stride)
            feat_ref[base + i, :, :, :w_stride] = v

    out_ref[...] = jnp.log(wav_ref[...]) * (1.0 / 20.0)


def kernel(w1, b1, w2, b2, x, wav):
    B, Cin, H, W = x.shape
    C1, C2 = w1.shape[0], w2.shape[0]
    H2, W2 = H - 2 * (_K - 1), W - 2 * (_K - 1)
    n = H * W
    keep = H2 * W
    Nw = wav.shape[-1]
    S = next(s for s in (16, 8, 4, 2, 1) if B % s == 0)

    K1 = _K * (_K * Cin + 1)
    C2p = -(-C2 // 8) * 8

    x_flat = x.reshape(B, Cin, n)
    wav3 = wav.reshape(B // S, S, Nw)
    w1s = jnp.transpose(w1, (3, 0, 2, 1)).reshape(_K, C1, _K * Cin)
    w1s = jnp.pad(w1s, ((0, 0), (0, 0), (0, 1)))
    w1cat = jnp.transpose(w1s, (1, 0, 2)).reshape(C1, K1).astype(jnp.bfloat16)
    w2s = jnp.transpose(w2, (3, 0, 2, 1)).reshape(_K, C2, _K * C1)
    w2s = jnp.pad(w2s, ((0, 0), (0, C2p - C2), (0, 0)))
    w2cat = w2s.reshape(_K * C2p, _K * C1).astype(jnp.bfloat16)
    b1c = b1.reshape(C1, 1)
    b2c = b2.reshape(C2, 1)

    kern = functools.partial(_fwd_kernel, s=S, n=n, w_stride=W, keep=keep,
                             c2=C2, c2p=C2p)

    out, feat_rows = pl.pallas_call(
        kern,
        grid=(B // S,),
        in_specs=[
            pl.BlockSpec((S, Cin, n), lambda b: (b, 0, 0)),
            pl.BlockSpec((C1, K1), lambda b: (0, 0)),
            pl.BlockSpec((C1, 1), lambda b: (0, 0)),
            pl.BlockSpec((_K * C2p, _K * C1), lambda b: (0, 0)),
            pl.BlockSpec((C2, 1), lambda b: (0, 0)),
            pl.BlockSpec((1, S, Nw), lambda b: (b, 0, 0)),
        ],
        out_specs=[
            pl.BlockSpec((1, S, Nw), lambda b: (b, 0, 0)),
            pl.BlockSpec((S, C2, H2, 128), lambda b: (b, 0, 0, 0)),
        ],
        out_shape=[
            jax.ShapeDtypeStruct((B // S, S, Nw), wav.dtype),
            jax.ShapeDtypeStruct((B, C2, H2, 128), jnp.float32),
        ],
        compiler_params=pltpu.CompilerParams(
            dimension_semantics=("parallel",)),
    )(x_flat, w1cat, b1c, w2cat, b2c, wav3)

    return out.reshape(B, Nw), feat_rows[:, :, :, :W2]
